# baseline (speedup 1.0000x reference)
import functools

import jax
import jax.numpy as jnp
from jax import lax
from jax.experimental import pallas as pl
from jax.experimental.pallas import tpu as pltpu
from jax.experimental.pallas import tpu_sc as plsc

B = 4
L = 8192
D = 768
SIZES = (168, 88)
OFFS = (0, 168)


@functools.cache
def _build_sc_kernel():
    info = plsc.get_sparse_core_info()
    nw = info.num_cores * info.num_subcores
    rows_per_w = L // nw
    assert sum(SIZES) == rows_per_w

    mesh = plsc.VectorSubcoreMesh(core_axis_name="c", subcore_axis_name="s")

    @functools.partial(
        pl.kernel,
        mesh=mesh,
        out_type=jax.ShapeDtypeStruct((B, L, D), jnp.float32),
        scratch_types=[pltpu.VMEM((max(SIZES), D), jnp.float32)],
    )
    def k(emb_hbm, out_hbm, buf):
        wid = lax.axis_index("s") * info.num_cores + lax.axis_index("c")
        base = wid * rows_per_w
        for c in range(len(SIZES)):
            row = base + OFFS[c]
            sub = buf.at[pl.ds(0, SIZES[c])]
            pltpu.sync_copy(emb_hbm.at[pl.ds(row, SIZES[c])], sub)
            for b in range(B):
                pltpu.sync_copy(sub, out_hbm.at[b, pl.ds(row, SIZES[c])])

    return k


def kernel(x, Embed):
    return _build_sc_kernel()(Embed)
